# baseline (speedup 1.0000x reference)
import functools

import jax
import jax.numpy as jnp
from jax import lax
from jax.experimental import pallas as pl
from jax.experimental.pallas import tpu as pltpu
from jax.experimental.pallas import tpu_sc as plsc

N_CLEAR = 4096
N_RAIN = 8192
D = 512

BI = 1024
BJ = 1024
BM = 1024

_SC_CORES = 2
_SC_SUBCORES = 16
_SC_WORKERS = _SC_CORES * _SC_SUBCORES
_ROWS_PER_WORKER = N_CLEAR // _SC_WORKERS


_BIG = 2**30


def _sq_block(x, y):
    x2 = jnp.sum(x * x, axis=1, keepdims=True)
    y2 = jnp.sum(y * y, axis=1)[None, :]
    dot2 = lax.dot_general(-2.0 * x, y, (((1,), (1,)), ((), ())),
                           preferred_element_type=jnp.float32)
    return (x2 + y2) + dot2


def _minsq_body(x_ref, y_ref, m_ref, mrun_ref):
    j = pl.program_id(1)
    nj = pl.num_programs(1)
    sq = _sq_block(x_ref[...], y_ref[...])
    mj = jnp.min(sq, axis=1, keepdims=True)

    @pl.when(j == 0)
    def _init():
        mrun_ref[...] = mj

    @pl.when(j > 0)
    def _acc():
        mrun_ref[...] = jnp.minimum(mrun_ref[...], mj)

    @pl.when(j == nj - 1)
    def _emit():
        m_ref[...] = mrun_ref[...]


def _extract_body(x_ref, y_ref, m_ref, idx_ref, hi_ref, bidx_ref):
    j = pl.program_id(1)
    nj = pl.num_programs(1)

    @pl.when(j == 0)
    def _init():
        m = m_ref[...]
        v = jnp.sqrt(jnp.maximum(m, 0.0))
        c = v * v
        cb = lax.bitcast_convert_type(c, jnp.int32)
        hi = c
        for k in range(1, 8):
            tk = lax.bitcast_convert_type(cb + k, jnp.float32)
            hi = jnp.where(jnp.sqrt(tk) == v, tk, hi)
        hi_ref[...] = jnp.where(v > 0.0, hi, 0.0)
        bidx_ref[...] = jnp.full((BI, 1), _BIG, jnp.int32)

    sq = _sq_block(x_ref[...], y_ref[...])
    cols = lax.broadcasted_iota(jnp.int32, (1, BJ), 1)
    lidx = jnp.min(jnp.where(sq <= hi_ref[...], cols, _BIG),
                   axis=1, keepdims=True)
    gidx = jnp.where(lidx < _BIG, lidx + j * BJ, _BIG)
    bidx_ref[...] = jnp.minimum(bidx_ref[...], gidx)

    @pl.when(j == nj - 1)
    def _emit():
        idx_ref[...] = bidx_ref[...]


def _nearest_idx(clear, rain):
    grid = (N_CLEAR // BI, N_RAIN // BJ)
    xy_specs = [
        pl.BlockSpec((BI, D), lambda i, j: (i, 0)),
        pl.BlockSpec((BJ, D), lambda i, j: (j, 0)),
    ]
    col_spec = pl.BlockSpec((BI, 1), lambda i, j: (i, 0))
    minsq = pl.pallas_call(
        _minsq_body,
        grid=grid,
        in_specs=xy_specs,
        out_specs=col_spec,
        out_shape=jax.ShapeDtypeStruct((N_CLEAR, 1), jnp.float32),
        scratch_shapes=[pltpu.VMEM((BI, 1), jnp.float32)],
    )(clear, rain)
    return pl.pallas_call(
        _extract_body,
        grid=grid,
        in_specs=xy_specs + [col_spec],
        out_specs=col_spec,
        out_shape=jax.ShapeDtypeStruct((N_CLEAR, 1), jnp.int32),
        scratch_shapes=[
            pltpu.VMEM((BI, 1), jnp.float32),
            pltpu.VMEM((BI, 1), jnp.int32),
        ],
    )(clear, rain, minsq)


@functools.partial(
    pl.kernel,
    mesh=plsc.VectorSubcoreMesh(core_axis_name="c", subcore_axis_name="s"),
    out_type=jax.ShapeDtypeStruct((N_CLEAR, D), jnp.float32),
    scratch_types=[
        pltpu.VMEM((_ROWS_PER_WORKER,), jnp.int32),
        pltpu.VMEM((_ROWS_PER_WORKER, D), jnp.float32),
        pltpu.SemaphoreType.DMA,
    ],
)
def _sc_gather(table_hbm, idx_hbm, out_hbm, idx_v, rows_v, sem):
    wid = lax.axis_index("s") * _SC_CORES + lax.axis_index("c")
    base = wid * _ROWS_PER_WORKER
    pltpu.sync_copy(idx_hbm.at[pl.ds(base, _ROWS_PER_WORKER)], idx_v)
    pltpu.async_copy(table_hbm.at[idx_v], rows_v, sem).wait()
    pltpu.sync_copy(rows_v, out_hbm.at[pl.ds(base, _ROWS_PER_WORKER)])


def _mlp_body(x_ref, a_ref, w1_ref, b1_ref, w2_ref, b2_ref, out_ref):
    x = x_ref[...]
    a = a_ref[...]
    comb = jnp.concatenate([x, a], axis=1)
    h = jax.nn.relu(lax.dot_general(comb, w1_ref[...],
                                    (((1,), (0,)), ((), ())),
                                    preferred_element_type=jnp.float32)
                    + b1_ref[...])
    s = lax.dot_general(h, w2_ref[...], (((1,), (0,)), ((), ())),
                        preferred_element_type=jnp.float32) + b2_ref[...]
    w = jax.nn.sigmoid(s)
    out_ref[...] = w * x + (1.0 - w) * a


def _mlp_fuse(clear, aligned, W1, b1, W2, b2):
    grid = (N_CLEAR // BM,)
    return pl.pallas_call(
        _mlp_body,
        grid=grid,
        in_specs=[
            pl.BlockSpec((BM, D), lambda i: (i, 0)),
            pl.BlockSpec((BM, D), lambda i: (i, 0)),
            pl.BlockSpec((2 * D, D), lambda i: (0, 0)),
            pl.BlockSpec((1, D), lambda i: (0, 0)),
            pl.BlockSpec((D, 1), lambda i: (0, 0)),
            pl.BlockSpec((1, 1), lambda i: (0, 0)),
        ],
        out_specs=pl.BlockSpec((BM, D), lambda i: (i, 0)),
        out_shape=jax.ShapeDtypeStruct((N_CLEAR, D), jnp.float32),
    )(clear, aligned, W1, b1.reshape(1, D), W2, b2.reshape(1, 1))


def kernel(clear_feature, rain_feature, W1, b1, W2, b2):
    idx = _nearest_idx(clear_feature, rain_feature).reshape(N_CLEAR)
    aligned = _sc_gather(rain_feature, idx)
    return _mlp_fuse(clear_feature, aligned, W1, b1, W2, b2)


# speedup vs baseline: 1.1938x; 1.1938x over previous
import functools

import jax
import jax.numpy as jnp
from jax import lax
from jax.experimental import pallas as pl
from jax.experimental.pallas import tpu as pltpu
from jax.experimental.pallas import tpu_sc as plsc

N_CLEAR = 4096
N_RAIN = 8192
D = 512

BI = 1024
BJ = 1024
BM = 1024

_SC_CORES = 2
_SC_SUBCORES = 16
_SC_WORKERS = _SC_CORES * _SC_SUBCORES
_ROWS_PER_WORKER = N_CLEAR // _SC_WORKERS


_BIG = 2**30


def _sq_block(x, y):
    x2 = jnp.sum(x * x, axis=1, keepdims=True)
    y2 = jnp.sum(y * y, axis=1)[None, :]
    dot2 = lax.dot_general(-2.0 * x, y, (((1,), (1,)), ((), ())),
                           preferred_element_type=jnp.float32)
    return (x2 + y2) + dot2


def _argmin_body(x_ref, y_ref, idx_ref, bv_ref, bi_ref):
    j = pl.program_id(1)
    nj = pl.num_programs(1)

    @pl.when(j == 0)
    def _init():
        bv_ref[...] = jnp.full((BI, 1), jnp.inf, jnp.float32)
        bi_ref[...] = jnp.zeros((BI, 1), jnp.int32)

    sq = _sq_block(x_ref[...], y_ref[...])
    dist = jnp.sqrt(jnp.maximum(sq, 0.0))

    minv = jnp.min(dist, axis=1, keepdims=True)
    cols = lax.broadcasted_iota(jnp.int32, (1, BJ), 1)
    lidx = jnp.min(jnp.where(dist == minv, cols, _BIG),
                   axis=1, keepdims=True)

    better = minv < bv_ref[...]
    bv_ref[...] = jnp.where(better, minv, bv_ref[...])
    bi_ref[...] = jnp.where(better, lidx + j * BJ, bi_ref[...])

    @pl.when(j == nj - 1)
    def _emit():
        idx_ref[...] = bi_ref[...]


def _nearest_idx(clear, rain):
    grid = (N_CLEAR // BI, N_RAIN // BJ)
    return pl.pallas_call(
        _argmin_body,
        grid=grid,
        in_specs=[
            pl.BlockSpec((BI, D), lambda i, j: (i, 0)),
            pl.BlockSpec((BJ, D), lambda i, j: (j, 0)),
        ],
        out_specs=pl.BlockSpec((BI, 1), lambda i, j: (i, 0)),
        out_shape=jax.ShapeDtypeStruct((N_CLEAR, 1), jnp.int32),
        scratch_shapes=[
            pltpu.VMEM((BI, 1), jnp.float32),
            pltpu.VMEM((BI, 1), jnp.int32),
        ],
    )(clear, rain)


@functools.partial(
    pl.kernel,
    mesh=plsc.VectorSubcoreMesh(core_axis_name="c", subcore_axis_name="s"),
    out_type=jax.ShapeDtypeStruct((N_CLEAR, D), jnp.float32),
    scratch_types=[
        pltpu.VMEM((_ROWS_PER_WORKER,), jnp.int32),
        pltpu.VMEM((_ROWS_PER_WORKER, D), jnp.float32),
        pltpu.SemaphoreType.DMA,
    ],
)
def _sc_gather(table_hbm, idx_hbm, out_hbm, idx_v, rows_v, sem):
    wid = lax.axis_index("s") * _SC_CORES + lax.axis_index("c")
    base = wid * _ROWS_PER_WORKER
    pltpu.sync_copy(idx_hbm.at[pl.ds(base, _ROWS_PER_WORKER)], idx_v)
    pltpu.async_copy(table_hbm.at[idx_v], rows_v, sem).wait()
    pltpu.sync_copy(rows_v, out_hbm.at[pl.ds(base, _ROWS_PER_WORKER)])


def _mlp_body(x_ref, a_ref, w1_ref, b1_ref, w2_ref, b2_ref, out_ref):
    x = x_ref[...]
    a = a_ref[...]
    comb = jnp.concatenate([x, a], axis=1)
    h = jax.nn.relu(lax.dot_general(comb, w1_ref[...],
                                    (((1,), (0,)), ((), ())),
                                    preferred_element_type=jnp.float32)
                    + b1_ref[...])
    s = lax.dot_general(h, w2_ref[...], (((1,), (0,)), ((), ())),
                        preferred_element_type=jnp.float32) + b2_ref[...]
    w = jax.nn.sigmoid(s)
    out_ref[...] = w * x + (1.0 - w) * a


def _mlp_fuse(clear, aligned, W1, b1, W2, b2):
    grid = (N_CLEAR // BM,)
    return pl.pallas_call(
        _mlp_body,
        grid=grid,
        in_specs=[
            pl.BlockSpec((BM, D), lambda i: (i, 0)),
            pl.BlockSpec((BM, D), lambda i: (i, 0)),
            pl.BlockSpec((2 * D, D), lambda i: (0, 0)),
            pl.BlockSpec((1, D), lambda i: (0, 0)),
            pl.BlockSpec((D, 1), lambda i: (0, 0)),
            pl.BlockSpec((1, 1), lambda i: (0, 0)),
        ],
        out_specs=pl.BlockSpec((BM, D), lambda i: (i, 0)),
        out_shape=jax.ShapeDtypeStruct((N_CLEAR, D), jnp.float32),
    )(clear, aligned, W1, b1.reshape(1, D), W2, b2.reshape(1, 1))


def kernel(clear_feature, rain_feature, W1, b1, W2, b2):
    idx = _nearest_idx(clear_feature, rain_feature).reshape(N_CLEAR)
    aligned = _sc_gather(rain_feature, idx)
    return _mlp_fuse(clear_feature, aligned, W1, b1, W2, b2)


# speedup vs baseline: 1.2344x; 1.0340x over previous
import functools

import jax
import jax.numpy as jnp
from jax import lax
from jax.experimental import pallas as pl
from jax.experimental.pallas import tpu as pltpu
from jax.experimental.pallas import tpu_sc as plsc

N_CLEAR = 4096
N_RAIN = 8192
D = 512

BI = 1024
BJ = 2048
BM = 1024

_SC_CORES = 2
_SC_SUBCORES = 16
_SC_WORKERS = _SC_CORES * _SC_SUBCORES
_ROWS_PER_WORKER = N_CLEAR // _SC_WORKERS


_BIG = 2**30


def _sq_block(x, y):
    x2 = jnp.sum(x * x, axis=1, keepdims=True)
    y2 = jnp.sum(y * y, axis=1)[None, :]
    dot2 = lax.dot_general(-2.0 * x, y, (((1,), (1,)), ((), ())),
                           preferred_element_type=jnp.float32)
    return (x2 + y2) + dot2


def _argmin_body(x_ref, y_ref, idx_ref, bv_ref, bi_ref):
    j = pl.program_id(1)
    nj = pl.num_programs(1)

    @pl.when(j == 0)
    def _init():
        bv_ref[...] = jnp.full((BI, 1), jnp.inf, jnp.float32)
        bi_ref[...] = jnp.zeros((BI, 1), jnp.int32)

    sq = _sq_block(x_ref[...], y_ref[...])
    dist = jnp.sqrt(jnp.maximum(sq, 0.0))

    minv = jnp.min(dist, axis=1, keepdims=True)
    cols = lax.broadcasted_iota(jnp.int32, (1, BJ), 1)
    lidx = jnp.min(jnp.where(dist == minv, cols, _BIG),
                   axis=1, keepdims=True)

    better = minv < bv_ref[...]
    bv_ref[...] = jnp.where(better, minv, bv_ref[...])
    bi_ref[...] = jnp.where(better, lidx + j * BJ, bi_ref[...])

    @pl.when(j == nj - 1)
    def _emit():
        idx_ref[...] = bi_ref[...]


def _nearest_idx(clear, rain):
    grid = (N_CLEAR // BI, N_RAIN // BJ)
    return pl.pallas_call(
        _argmin_body,
        grid=grid,
        in_specs=[
            pl.BlockSpec((BI, D), lambda i, j: (i, 0)),
            pl.BlockSpec((BJ, D), lambda i, j: (j, 0)),
        ],
        out_specs=pl.BlockSpec((BI, 1), lambda i, j: (i, 0)),
        out_shape=jax.ShapeDtypeStruct((N_CLEAR, 1), jnp.int32),
        scratch_shapes=[
            pltpu.VMEM((BI, 1), jnp.float32),
            pltpu.VMEM((BI, 1), jnp.int32),
        ],
    )(clear, rain)


@functools.partial(
    pl.kernel,
    mesh=plsc.VectorSubcoreMesh(core_axis_name="c", subcore_axis_name="s"),
    out_type=jax.ShapeDtypeStruct((N_CLEAR, D), jnp.float32),
    scratch_types=[
        pltpu.VMEM((_ROWS_PER_WORKER,), jnp.int32),
        pltpu.VMEM((_ROWS_PER_WORKER, D), jnp.float32),
        pltpu.SemaphoreType.DMA,
    ],
)
def _sc_gather(table_hbm, idx_hbm, out_hbm, idx_v, rows_v, sem):
    wid = lax.axis_index("s") * _SC_CORES + lax.axis_index("c")
    base = wid * _ROWS_PER_WORKER
    pltpu.sync_copy(idx_hbm.at[pl.ds(base, _ROWS_PER_WORKER)], idx_v)
    pltpu.async_copy(table_hbm.at[idx_v], rows_v, sem).wait()
    pltpu.sync_copy(rows_v, out_hbm.at[pl.ds(base, _ROWS_PER_WORKER)])


def _mlp_body(x_ref, a_ref, w1_ref, b1_ref, w2_ref, b2_ref, out_ref):
    x = x_ref[...]
    a = a_ref[...]
    comb = jnp.concatenate([x, a], axis=1)
    h = jax.nn.relu(lax.dot_general(comb, w1_ref[...],
                                    (((1,), (0,)), ((), ())),
                                    preferred_element_type=jnp.float32)
                    + b1_ref[...])
    s = lax.dot_general(h, w2_ref[...], (((1,), (0,)), ((), ())),
                        preferred_element_type=jnp.float32) + b2_ref[...]
    w = jax.nn.sigmoid(s)
    out_ref[...] = w * x + (1.0 - w) * a


def _mlp_fuse(clear, aligned, W1, b1, W2, b2):
    grid = (N_CLEAR // BM,)
    return pl.pallas_call(
        _mlp_body,
        grid=grid,
        in_specs=[
            pl.BlockSpec((BM, D), lambda i: (i, 0)),
            pl.BlockSpec((BM, D), lambda i: (i, 0)),
            pl.BlockSpec((2 * D, D), lambda i: (0, 0)),
            pl.BlockSpec((1, D), lambda i: (0, 0)),
            pl.BlockSpec((D, 1), lambda i: (0, 0)),
            pl.BlockSpec((1, 1), lambda i: (0, 0)),
        ],
        out_specs=pl.BlockSpec((BM, D), lambda i: (i, 0)),
        out_shape=jax.ShapeDtypeStruct((N_CLEAR, D), jnp.float32),
    )(clear, aligned, W1, b1.reshape(1, D), W2, b2.reshape(1, 1))


def kernel(clear_feature, rain_feature, W1, b1, W2, b2):
    idx = _nearest_idx(clear_feature, rain_feature).reshape(N_CLEAR)
    aligned = _sc_gather(rain_feature, idx)
    return _mlp_fuse(clear_feature, aligned, W1, b1, W2, b2)


# speedup vs baseline: 1.3193x; 1.0688x over previous
import functools

import jax
import jax.numpy as jnp
from jax import lax
from jax.experimental import pallas as pl
from jax.experimental.pallas import tpu as pltpu
from jax.experimental.pallas import tpu_sc as plsc

N_CLEAR = 4096
N_RAIN = 8192
D = 512

BI = 1024
BJ = 2048
BM = 1024

_SC_CORES = 2
_SC_SUBCORES = 16
_SC_WORKERS = _SC_CORES * _SC_SUBCORES
_ROWS_PER_WORKER = N_CLEAR // _SC_WORKERS


_BIG = 2**30


def _sq_block(x, y):
    x2 = jnp.sum(x * x, axis=1, keepdims=True)
    y2 = jnp.sum(y * y, axis=1)[None, :]
    dot2 = lax.dot_general(-2.0 * x, y, (((1,), (1,)), ((), ())),
                           preferred_element_type=jnp.float32)
    return (x2 + y2) + dot2


_C = 128


def _argmin_body(x_ref, y_ref, idx_ref, vacc_ref, cacc_ref):
    j = pl.program_id(1)
    nj = pl.num_programs(1)

    @pl.when(j == 0)
    def _init():
        vacc_ref[...] = jnp.full((BI, _C), jnp.inf, jnp.float32)
        cacc_ref[...] = jnp.zeros((BI, _C), jnp.float32)

    sq = _sq_block(x_ref[...], y_ref[...])
    dist = jnp.sqrt(jnp.maximum(sq, 0.0))

    lane = lax.broadcasted_iota(jnp.int32, (1, _C), 1)
    vacc = vacc_ref[...]
    cacc = cacc_ref[...]
    for c in range(BJ // _C):
        dc = dist[:, c * _C:(c + 1) * _C]
        colf = (lane + (j * BJ + c * _C)).astype(jnp.float32)
        lt = dc < vacc
        vacc = jnp.minimum(dc, vacc)
        cacc = jnp.where(lt, colf, cacc)
    vacc_ref[...] = vacc
    cacc_ref[...] = cacc

    @pl.when(j == nj - 1)
    def _emit():
        mv = jnp.min(vacc, axis=1, keepdims=True)
        lidxf = jnp.min(jnp.where(vacc == mv, cacc, 3.0e38),
                        axis=1, keepdims=True)
        idx_ref[...] = lidxf.astype(jnp.int32)


def _nearest_idx(clear, rain):
    grid = (N_CLEAR // BI, N_RAIN // BJ)
    return pl.pallas_call(
        _argmin_body,
        grid=grid,
        in_specs=[
            pl.BlockSpec((BI, D), lambda i, j: (i, 0)),
            pl.BlockSpec((BJ, D), lambda i, j: (j, 0)),
        ],
        out_specs=pl.BlockSpec((BI, 1), lambda i, j: (i, 0)),
        out_shape=jax.ShapeDtypeStruct((N_CLEAR, 1), jnp.int32),
        scratch_shapes=[
            pltpu.VMEM((BI, _C), jnp.float32),
            pltpu.VMEM((BI, _C), jnp.float32),
        ],
    )(clear, rain)


@functools.partial(
    pl.kernel,
    mesh=plsc.VectorSubcoreMesh(core_axis_name="c", subcore_axis_name="s"),
    out_type=jax.ShapeDtypeStruct((N_CLEAR, D), jnp.float32),
    scratch_types=[
        pltpu.VMEM((_ROWS_PER_WORKER,), jnp.int32),
        pltpu.VMEM((_ROWS_PER_WORKER, D), jnp.float32),
        pltpu.SemaphoreType.DMA,
    ],
)
def _sc_gather(table_hbm, idx_hbm, out_hbm, idx_v, rows_v, sem):
    wid = lax.axis_index("s") * _SC_CORES + lax.axis_index("c")
    base = wid * _ROWS_PER_WORKER
    pltpu.sync_copy(idx_hbm.at[pl.ds(base, _ROWS_PER_WORKER)], idx_v)
    pltpu.async_copy(table_hbm.at[idx_v], rows_v, sem).wait()
    pltpu.sync_copy(rows_v, out_hbm.at[pl.ds(base, _ROWS_PER_WORKER)])


def _mlp_body(x_ref, a_ref, w1_ref, b1_ref, w2_ref, b2_ref, out_ref):
    x = x_ref[...]
    a = a_ref[...]
    comb = jnp.concatenate([x, a], axis=1)
    h = jax.nn.relu(lax.dot_general(comb, w1_ref[...],
                                    (((1,), (0,)), ((), ())),
                                    preferred_element_type=jnp.float32)
                    + b1_ref[...])
    s = lax.dot_general(h, w2_ref[...], (((1,), (0,)), ((), ())),
                        preferred_element_type=jnp.float32) + b2_ref[...]
    w = jax.nn.sigmoid(s)
    out_ref[...] = w * x + (1.0 - w) * a


def _mlp_fuse(clear, aligned, W1, b1, W2, b2):
    grid = (N_CLEAR // BM,)
    return pl.pallas_call(
        _mlp_body,
        grid=grid,
        in_specs=[
            pl.BlockSpec((BM, D), lambda i: (i, 0)),
            pl.BlockSpec((BM, D), lambda i: (i, 0)),
            pl.BlockSpec((2 * D, D), lambda i: (0, 0)),
            pl.BlockSpec((1, D), lambda i: (0, 0)),
            pl.BlockSpec((D, 1), lambda i: (0, 0)),
            pl.BlockSpec((1, 1), lambda i: (0, 0)),
        ],
        out_specs=pl.BlockSpec((BM, D), lambda i: (i, 0)),
        out_shape=jax.ShapeDtypeStruct((N_CLEAR, D), jnp.float32),
    )(clear, aligned, W1, b1.reshape(1, D), W2, b2.reshape(1, 1))


def kernel(clear_feature, rain_feature, W1, b1, W2, b2):
    idx = _nearest_idx(clear_feature, rain_feature).reshape(N_CLEAR)
    aligned = _sc_gather(rain_feature, idx)
    return _mlp_fuse(clear_feature, aligned, W1, b1, W2, b2)


# speedup vs baseline: 1.3802x; 1.0461x over previous
import functools

import jax
import jax.numpy as jnp
from jax import lax
from jax.experimental import pallas as pl
from jax.experimental.pallas import tpu as pltpu
from jax.experimental.pallas import tpu_sc as plsc

N_CLEAR = 4096
N_RAIN = 8192
D = 512

BI = 2048
BJ = 2048
BM = 1024

_SC_CORES = 2
_SC_SUBCORES = 16
_SC_WORKERS = _SC_CORES * _SC_SUBCORES
_ROWS_PER_WORKER = N_CLEAR // _SC_WORKERS


_BIG = 2**30


def _sq_parts(x, y):
    x2 = jnp.sum(x * x, axis=1, keepdims=True)
    y2 = jnp.sum(y * y, axis=1)[None, :]
    dot2 = lax.dot_general(-2.0 * x, y, (((1,), (1,)), ((), ())),
                           preferred_element_type=jnp.float32)
    return x2, y2, dot2


_C = 128


def _argmin_body(x_ref, y_ref, idx_ref, vacc_ref, cacc_ref):
    j = pl.program_id(1)
    nj = pl.num_programs(1)

    @pl.when(j == 0)
    def _init():
        vacc_ref[...] = jnp.full((BI, _C), jnp.inf, jnp.float32)
        cacc_ref[...] = jnp.zeros((BI, _C), jnp.float32)

    x2, y2, dot2 = _sq_parts(x_ref[...], y_ref[...])

    lane = lax.broadcasted_iota(jnp.int32, (1, _C), 1)
    vacc = vacc_ref[...]
    cacc = cacc_ref[...]
    for c in range(BJ // _C):
        cs = slice(c * _C, (c + 1) * _C)
        sq = (x2 + y2[:, cs]) + dot2[:, cs]
        dc = jnp.sqrt(jnp.maximum(sq, 0.0))
        colf = (lane + (j * BJ + c * _C)).astype(jnp.float32)
        lt = dc < vacc
        vacc = jnp.minimum(dc, vacc)
        cacc = jnp.where(lt, colf, cacc)
    vacc_ref[...] = vacc
    cacc_ref[...] = cacc

    @pl.when(j == nj - 1)
    def _emit():
        mv = jnp.min(vacc, axis=1, keepdims=True)
        lidxf = jnp.min(jnp.where(vacc == mv, cacc, 3.0e38),
                        axis=1, keepdims=True)
        idx_ref[...] = lidxf.astype(jnp.int32)


def _nearest_idx(clear, rain):
    grid = (N_CLEAR // BI, N_RAIN // BJ)
    return pl.pallas_call(
        _argmin_body,
        grid=grid,
        in_specs=[
            pl.BlockSpec((BI, D), lambda i, j: (i, 0)),
            pl.BlockSpec((BJ, D), lambda i, j: (j, 0)),
        ],
        out_specs=pl.BlockSpec((BI, 1), lambda i, j: (i, 0)),
        out_shape=jax.ShapeDtypeStruct((N_CLEAR, 1), jnp.int32),
        scratch_shapes=[
            pltpu.VMEM((BI, _C), jnp.float32),
            pltpu.VMEM((BI, _C), jnp.float32),
        ],
    )(clear, rain)


@functools.partial(
    pl.kernel,
    mesh=plsc.VectorSubcoreMesh(core_axis_name="c", subcore_axis_name="s"),
    out_type=jax.ShapeDtypeStruct((N_CLEAR, D), jnp.float32),
    scratch_types=[
        pltpu.VMEM((_ROWS_PER_WORKER,), jnp.int32),
        pltpu.VMEM((_ROWS_PER_WORKER, D), jnp.float32),
        pltpu.SemaphoreType.DMA,
    ],
)
def _sc_gather(table_hbm, idx_hbm, out_hbm, idx_v, rows_v, sem):
    wid = lax.axis_index("s") * _SC_CORES + lax.axis_index("c")
    base = wid * _ROWS_PER_WORKER
    pltpu.sync_copy(idx_hbm.at[pl.ds(base, _ROWS_PER_WORKER)], idx_v)
    pltpu.async_copy(table_hbm.at[idx_v], rows_v, sem).wait()
    pltpu.sync_copy(rows_v, out_hbm.at[pl.ds(base, _ROWS_PER_WORKER)])


def _mlp_body(x_ref, a_ref, w1_ref, b1_ref, w2_ref, b2_ref, out_ref):
    x = x_ref[...]
    a = a_ref[...]
    comb = jnp.concatenate([x, a], axis=1)
    h = jax.nn.relu(lax.dot_general(comb, w1_ref[...],
                                    (((1,), (0,)), ((), ())),
                                    preferred_element_type=jnp.float32)
                    + b1_ref[...])
    s = lax.dot_general(h, w2_ref[...], (((1,), (0,)), ((), ())),
                        preferred_element_type=jnp.float32) + b2_ref[...]
    w = jax.nn.sigmoid(s)
    out_ref[...] = w * x + (1.0 - w) * a


def _mlp_fuse(clear, aligned, W1, b1, W2, b2):
    grid = (N_CLEAR // BM,)
    return pl.pallas_call(
        _mlp_body,
        grid=grid,
        in_specs=[
            pl.BlockSpec((BM, D), lambda i: (i, 0)),
            pl.BlockSpec((BM, D), lambda i: (i, 0)),
            pl.BlockSpec((2 * D, D), lambda i: (0, 0)),
            pl.BlockSpec((1, D), lambda i: (0, 0)),
            pl.BlockSpec((D, 1), lambda i: (0, 0)),
            pl.BlockSpec((1, 1), lambda i: (0, 0)),
        ],
        out_specs=pl.BlockSpec((BM, D), lambda i: (i, 0)),
        out_shape=jax.ShapeDtypeStruct((N_CLEAR, D), jnp.float32),
    )(clear, aligned, W1, b1.reshape(1, D), W2, b2.reshape(1, 1))


def kernel(clear_feature, rain_feature, W1, b1, W2, b2):
    idx = _nearest_idx(clear_feature, rain_feature).reshape(N_CLEAR)
    aligned = _sc_gather(rain_feature, idx)
    return _mlp_fuse(clear_feature, aligned, W1, b1, W2, b2)
